# baseline (speedup 1.0000x reference)
import jax
import jax.numpy as jnp
from jax.experimental import pallas as pl


def _gates1_body(ru_ref, h_ref, rh_ref, u_ref):
    h = h_ref[...]
    hid = h.shape[1]
    ru = ru_ref[...]
    r = jax.nn.sigmoid(ru[:, :hid])
    u = jax.nn.sigmoid(ru[:, hid:])
    rh_ref[...] = r * h
    u_ref[...] = u


def _gates2_body(c_ref, u_ref, h_ref, out_ref):
    c = jnp.tanh(c_ref[...])
    u = u_ref[...]
    h = h_ref[...]
    out_ref[...] = u * h + (1.0 - u) * c


def _gates1(ru_logits, h):
    n, hid = h.shape
    return pl.pallas_call(
        _gates1_body,
        out_shape=[
            jax.ShapeDtypeStruct((n, hid), ru_logits.dtype),
            jax.ShapeDtypeStruct((n, hid), ru_logits.dtype),
        ],
    )(ru_logits, h)


def _gates2(c_logits, u, h):
    return pl.pallas_call(
        _gates2_body,
        out_shape=jax.ShapeDtypeStruct(h.shape, h.dtype),
    )(c_logits, u, h)


def kernel(x, edge_idx, edge_attr, Wr, br, Wu, bu, Wc, bc):
    T, N, IN = x.shape
    K, CIN, HID = Wr.shape
    E = edge_attr.shape[1]

    row = edge_idx[:, 0]
    col = edge_idx[:, 1]
    ew = jnp.where(row == col, 0.0, edge_attr)
    deg = jax.vmap(lambda r, w: jnp.zeros((N,), x.dtype).at[r].add(w))(row, ew)
    dis = jnp.where(deg > 0, deg ** -0.5, 0.0)
    w_hat = -jnp.take_along_axis(dis, row, 1) * ew * jnp.take_along_axis(dis, col, 1)

    Wru = jnp.concatenate([Wr, Wu], axis=-1)
    Wru_x = Wru[:, :IN, :]
    Wru_h = Wru[:, IN:, :]
    Wc_x = Wc[:, :IN, :]
    Wc_h = Wc[:, IN:, :]

    xp_ru = jnp.einsum('tni,kio->ktno', x, Wru_x)
    xp_c = jnp.einsum('tni,kio->ktno', x, Wc_x)

    b_ru = jnp.concatenate([br, bu])

    def lap(v, r_t, c_t, w_t):
        return jnp.zeros_like(v).at[c_t].add(v[r_t] * w_t[:, None])

    h = jnp.zeros((N, HID), dtype=x.dtype)
    for t in range(T):
        r_t, c_t, w_t = row[t], col[t], w_hat[t]
        y = xp_ru[:, t] + jnp.einsum('nh,kho->kno', h, Wru_h)
        p1 = lap(y[2], r_t, c_t, w_t)
        z = y[1] + 2.0 * p1
        p2 = lap(z, r_t, c_t, w_t)
        ru_logits = y[0] - y[2] + p2 + b_ru
        rh, u = _gates1(ru_logits, h)

        yc = xp_c[:, t] + jnp.einsum('nh,kho->kno', rh, Wc_h)
        q1 = lap(yc[2], r_t, c_t, w_t)
        zc = yc[1] + 2.0 * q1
        q2 = lap(zc, r_t, c_t, w_t)
        c_logits = yc[0] - yc[2] + q2 + bc
        h = _gates2(c_logits, u, h)
    return h


# speedup vs baseline: 1.3626x; 1.3626x over previous
import functools
import jax
import jax.numpy as jnp
from jax import lax
from jax.experimental import pallas as pl
from jax.experimental.pallas import tpu as pltpu, tpu_sc as plsc

_NC = 2
_NS = 16
_NW = _NC * _NS
_CH = 128


@functools.lru_cache(maxsize=None)
def _lap_sc_call(n_nodes, width, n_chunks):
    mesh = plsc.VectorSubcoreMesh(
        core_axis_name="c", subcore_axis_name="s",
        num_cores=_NC, num_subcores=_NS)
    rps = n_nodes // _NS
    zr = 128
    n_z = rps // zr

    @functools.partial(
        pl.kernel,
        mesh=mesh,
        compiler_params=pltpu.CompilerParams(use_tc_tiling_on_sc=False),
        out_type=jax.ShapeDtypeStruct((_NC, n_nodes, width), jnp.float32),
        scratch_types=[
            pltpu.VMEM((n_chunks, _CH), jnp.int32),
            pltpu.VMEM((n_chunks, _CH), jnp.int32),
            pltpu.VMEM((n_chunks, _CH), jnp.float32),
            pltpu.VMEM((_CH, width), jnp.float32),
            pltpu.VMEM((zr, width), jnp.float32),
            pltpu.VMEM_SHARED((n_nodes, width), jnp.float32),
            pltpu.SemaphoreType.DMA,
        ],
    )
    def k(v_hbm, row_hbm, col_hbm, w_hbm, out_hbm,
          row_b, col_b, w_b, rows_v, zbuf, acc, sem):
        cid = lax.axis_index("c")
        sid = lax.axis_index("s")
        wid = sid * _NC + cid

        zero16 = jnp.zeros((16,), jnp.float32)

        def _zrow(i, carry):
            for u in range(width // 16):
                zbuf[i, pl.ds(u * 16, 16)] = zero16
            return carry
        lax.fori_loop(0, zr, _zrow, 0)

        for b in range(n_z):
            pltpu.sync_copy(zbuf, acc.at[pl.ds(sid * rps + b * zr, zr)])
        plsc.subcore_barrier()

        pltpu.sync_copy(row_hbm.at[wid], row_b)
        pltpu.sync_copy(col_hbm.at[wid], col_b)
        pltpu.sync_copy(w_hbm.at[wid], w_b)

        def _chunk(j, carry):
            pltpu.async_copy(v_hbm.at[row_b.at[j]], rows_v, sem).wait()

            def _scale(g, c2):
                w16 = w_b[j, pl.ds(g * 16, 16)]
                for i in range(16):
                    e = g * 16 + i
                    ws = w16[i]
                    for u in range(width // 16):
                        sl = pl.ds(u * 16, 16)
                        rows_v[e, sl] = rows_v[e, sl] * ws
                return c2
            lax.fori_loop(0, _CH // 16, _scale, 0)

            pltpu.sync_copy(rows_v, acc.at[col_b.at[j]], add=True)
            return carry
        lax.fori_loop(0, n_chunks, _chunk, 0)

        plsc.subcore_barrier()

        for b in range(n_z):
            sl = pl.ds(sid * rps + b * zr, zr)
            pltpu.sync_copy(acc.at[sl], out_hbm.at[cid, sl])

    return k


def _lap_sc(v, row_c, col_c, w_c):
    n, width = v.shape
    parts = _lap_sc_call(n, width, row_c.shape[1])(v, row_c, col_c, w_c)
    return parts[0] + parts[1]


def _gates1_body(ru_ref, h_ref, rh_ref, u_ref):
    h = h_ref[...]
    hid = h.shape[1]
    ru = ru_ref[...]
    rh_ref[...] = jax.nn.sigmoid(ru[:, :hid]) * h
    u_ref[...] = jax.nn.sigmoid(ru[:, hid:])


def _gates2_body(c_ref, u_ref, h_ref, out_ref):
    c = jnp.tanh(c_ref[...])
    u = u_ref[...]
    h = h_ref[...]
    out_ref[...] = u * h + (1.0 - u) * c


def _gates1(ru_logits, h):
    n, hid = h.shape
    return pl.pallas_call(
        _gates1_body,
        out_shape=[
            jax.ShapeDtypeStruct((n, hid), ru_logits.dtype),
            jax.ShapeDtypeStruct((n, hid), ru_logits.dtype),
        ],
    )(ru_logits, h)


def _gates2(c_logits, u, h):
    return pl.pallas_call(
        _gates2_body,
        out_shape=jax.ShapeDtypeStruct(h.shape, h.dtype),
    )(c_logits, u, h)


def kernel(x, edge_idx, edge_attr, Wr, br, Wu, bu, Wc, bc):
    T, N, IN = x.shape
    K, CIN, HID = Wr.shape
    E = edge_attr.shape[1]

    row = edge_idx[:, 0]
    col = edge_idx[:, 1]
    ew = jnp.where(row == col, 0.0, edge_attr)
    deg = jax.vmap(lambda r, w: jnp.zeros((N,), x.dtype).at[r].add(w))(row, ew)
    dis = jnp.where(deg > 0, deg ** -0.5, 0.0)
    w_hat = -jnp.take_along_axis(dis, row, 1) * ew * jnp.take_along_axis(dis, col, 1)

    n_chunks = -(-E // (_NW * _CH))
    e_pad = _NW * _CH * n_chunks
    pad = [(0, 0), (0, e_pad - E)]
    row_c = jnp.pad(row, pad).reshape(T, _NW, n_chunks, _CH)
    col_c = jnp.pad(col, pad).reshape(T, _NW, n_chunks, _CH)
    w_c = jnp.pad(w_hat, pad).reshape(T, _NW, n_chunks, _CH)

    Wru = jnp.concatenate([Wr, Wu], axis=-1)
    Wru_x, Wru_h = Wru[:, :IN, :], Wru[:, IN:, :]
    Wc_x, Wc_h = Wc[:, :IN, :], Wc[:, IN:, :]

    xp_ru = jnp.einsum('tni,kio->ktno', x, Wru_x)
    xp_c = jnp.einsum('tni,kio->ktno', x, Wc_x)

    b_ru = jnp.concatenate([br, bu])

    n_pad = -(-N // (_NS * 128)) * (_NS * 128)
    npad = [(0, 0), (0, 0), (0, n_pad - N), (0, 0)]
    xp_ru = jnp.pad(xp_ru, npad)
    xp_c = jnp.pad(xp_c, npad)

    h = jnp.zeros((n_pad, HID), dtype=x.dtype)
    for t in range(T):
        rc, cc, wc = row_c[t], col_c[t], w_c[t]
        y = xp_ru[:, t] + jnp.einsum('nh,kho->kno', h, Wru_h)
        p1 = _lap_sc(y[2], rc, cc, wc)
        z = y[1] + 2.0 * p1
        p2 = _lap_sc(z, rc, cc, wc)
        ru_logits = y[0] - y[2] + p2 + b_ru
        rh, u = _gates1(ru_logits, h)

        yc = xp_c[:, t] + jnp.einsum('nh,kho->kno', rh, Wc_h)
        q1 = _lap_sc(yc[2], rc, cc, wc)
        zc = yc[1] + 2.0 * q1
        q2 = _lap_sc(zc, rc, cc, wc)
        c_logits = yc[0] - yc[2] + q2 + bc
        h = _gates2(c_logits, u, h)
    return h[:N]
